# Initial kernel scaffold; baseline (speedup 1.0000x reference)
import jax
import jax.numpy as jnp
from jax.experimental import pallas as pl


def kernel(x, table, W, b):
    raise NotImplementedError("write your pallas kernel here")


# speedup vs baseline: 8.6174x; 8.6174x over previous
import functools

import jax
import jax.numpy as jnp
from jax import lax
from jax.experimental import pallas as pl
from jax.experimental.pallas import tpu as pltpu
from jax.experimental.pallas import tpu_sc as plsc

VOCAB = 1000000
EMBED_DIM = 32
SEQ_LEN = 50
BATCH = 4096
OUT_DIM = 32
N_TOKENS = BATCH * SEQ_LEN

NC = 2
NS = 16
NW = NC * NS

CHUNK = 128
ROWS_PER_W = N_TOKENS // NW
CHUNKS_PER_W = ROWS_PER_W // CHUNK


def _sc_gather(idx3d, table):
    mesh = plsc.VectorSubcoreMesh(
        core_axis_name="c", subcore_axis_name="s", num_cores=NC, num_subcores=NS
    )

    @functools.partial(
        pl.kernel,
        out_type=jax.ShapeDtypeStruct((N_TOKENS, EMBED_DIM), jnp.float32),
        mesh=mesh,
        scratch_types=[
            pltpu.VMEM((CHUNKS_PER_W, CHUNK), jnp.int32),
            pltpu.VMEM((CHUNK, EMBED_DIM), jnp.float32),
            pltpu.SemaphoreType.DMA,
        ],
        compiler_params=pltpu.CompilerParams(use_tc_tiling_on_sc=False),
    )
    def gather_kernel(idx_hbm, table_hbm, out_hbm, idx_v, rows_v, sem):
        wid = lax.axis_index("s") * NC + lax.axis_index("c")
        row_base = wid * ROWS_PER_W
        pltpu.sync_copy(idx_hbm.at[wid], idx_v)

        def body(j, carry):
            pltpu.async_copy(table_hbm.at[idx_v.at[j]], rows_v, sem).wait()
            pltpu.sync_copy(rows_v, out_hbm.at[pl.ds(row_base + j * CHUNK, CHUNK)])
            return carry

        lax.fori_loop(0, CHUNKS_PER_W, body, 0)

    return gather_kernel(idx3d, table)


def _tc_matmul(g, W, b2d):
    BB = 512
    in_feat = SEQ_LEN * EMBED_DIM

    def mm_kernel(g_ref, w_ref, b_ref, o_ref):
        acc = lax.dot_general(
            g_ref[...],
            w_ref[...],
            (((1,), (1,)), ((), ())),
            preferred_element_type=jnp.float32,
        )
        o_ref[...] = acc + b_ref[...]

    return pl.pallas_call(
        mm_kernel,
        grid=(BATCH // BB,),
        in_specs=[
            pl.BlockSpec((BB, in_feat), lambda i: (i, 0)),
            pl.BlockSpec((OUT_DIM, in_feat), lambda i: (0, 0)),
            pl.BlockSpec((1, OUT_DIM), lambda i: (0, 0)),
        ],
        out_specs=pl.BlockSpec((BB, OUT_DIM), lambda i: (i, 0)),
        out_shape=jax.ShapeDtypeStruct((BATCH, OUT_DIM), jnp.float32),
    )(g, W, b2d)


def kernel(x, table, W, b):
    idx3d = x.astype(jnp.int32).reshape(NW, CHUNKS_PER_W, CHUNK)
    gathered = _sc_gather(idx3d, table)
    g = gathered.reshape(BATCH, SEQ_LEN * EMBED_DIM)
    return _tc_matmul(g, W, b.reshape(1, OUT_DIM))
